# Initial kernel scaffold; baseline (speedup 1.0000x reference)
import jax
import jax.numpy as jnp
from jax.experimental import pallas as pl


def kernel(x, batch, y, W0, b0, W1, b1, W2, b2):
    raise NotImplementedError("write your pallas kernel here")


# speedup vs baseline: 5.1353x; 5.1353x over previous
import jax
import jax.numpy as jnp
from jax.experimental import pallas as pl
from jax.experimental.pallas import tpu as pltpu

NUM_SEGS = 512
ROWS = 100000
DIM = 128
BLK = 2000
NBLK = ROWS // BLK


def _seg_mlp_kernel(batch_ref, x_ref, w0_ref, b0_ref, w1_ref, b1_ref,
                    w2_ref, b2_ref, out_ref, acc_ref):
    i = pl.program_id(0)

    @pl.when(i == 0)
    def _init():
        acc_ref[...] = jnp.zeros_like(acc_ref)

    b = batch_ref[0, 0, :]
    x = x_ref[...]
    seg_ids = jax.lax.broadcasted_iota(jnp.int32, (BLK, NUM_SEGS), 1)
    onehot = (seg_ids == b[:, None]).astype(jnp.float32)
    partial = jax.lax.dot_general(onehot, x, (((0,), (0,)), ((), ())),
                                  preferred_element_type=jnp.float32)
    acc_ref[...] += partial

    @pl.when(i == NBLK - 1)
    def _mlp():
        seg = acc_ref[...]
        h0 = jax.lax.dot_general(seg, w0_ref[...], (((1,), (1,)), ((), ())),
                                 preferred_element_type=jnp.float32)
        h0 = jnp.maximum(h0 + b0_ref[...], 0.0)
        h1 = jax.lax.dot_general(h0, w1_ref[...], (((1,), (1,)), ((), ())),
                                 preferred_element_type=jnp.float32)
        h1 = jnp.maximum(h1 + b1_ref[...], 0.0)
        h2 = jax.lax.dot_general(h1, w2_ref[...], (((1,), (1,)), ((), ())),
                                 preferred_element_type=jnp.float32)
        out_ref[...] = h2 + b2_ref[...]


def kernel(x, batch, y, W0, b0, W1, b1, W2, b2):
    batch3 = batch.astype(jnp.int32).reshape(NBLK, 1, BLK)
    pred = pl.pallas_call(
        _seg_mlp_kernel,
        grid=(NBLK,),
        in_specs=[
            pl.BlockSpec((1, 1, BLK), lambda i: (i, 0, 0)),
            pl.BlockSpec((BLK, DIM), lambda i: (i, 0)),
            pl.BlockSpec((64, DIM), lambda i: (0, 0)),
            pl.BlockSpec((1, 64), lambda i: (0, 0)),
            pl.BlockSpec((32, 64), lambda i: (0, 0)),
            pl.BlockSpec((1, 32), lambda i: (0, 0)),
            pl.BlockSpec((8, 32), lambda i: (0, 0)),
            pl.BlockSpec((1, 8), lambda i: (0, 0)),
        ],
        out_specs=pl.BlockSpec((NUM_SEGS, 8), lambda i: (0, 0)),
        out_shape=jax.ShapeDtypeStruct((NUM_SEGS, 8), jnp.float32),
        scratch_shapes=[pltpu.VMEM((NUM_SEGS, DIM), jnp.float32)],
    )(batch3, x, W0, b0.reshape(1, 64), W1, b1.reshape(1, 32),
      jnp.pad(W2, ((0, 7), (0, 0))), jnp.pad(b2.reshape(1, 1), ((0, 0), (0, 7))))
    return (pred[:, :1], y)
